# baseline (speedup 1.0000x reference)
import jax
import jax.numpy as jnp
from jax import lax
from jax.experimental import pallas as pl
from jax.experimental.pallas import tpu as pltpu
from jax.experimental.pallas import tpu_sc as plsc

B = 16384
D = 64
N = 1000000
NC = 2
NS = 16
NW = NC * NS
BPW = B // NW
L = 16
NBLK = (N + 127) // 128
LASTB = NBLK - 1
GOUT = B + NW
WLCAP = B + L


def _gather_body(uemb_t, iemb_t, ulast, ilast, uidx_f, iidx_f,
                 ug_hbm, ig_hbm,
                 ix_v, wl_r, wl_p, hit_p, off_v, blk0, blk1, blk2, blk3,
                 stg_v, pos_st,
                 sem, sem0, sem1, sem2, sem3):
    wid = lax.axis_index("s") * NC + lax.axis_index("c")
    lo_blk = lax.shift_right_logical(wid * NBLK, 5)
    hi_blk = lax.shift_right_logical((wid + 1) * NBLK, 5)
    lo_r = lo_blk * 128
    hi_r = hi_blk * 128
    lanes = lax.iota(jnp.int32, L)
    lane0 = lanes == 0
    trash = B + wid
    bufs = (blk0, blk1, blk2, blk3)
    sems = (sem0, sem1, sem2, sem3)

    for tab, last, idx_f, og in ((uemb_t, ulast, uidx_f, ug_hbm),
                                 (iemb_t, ilast, iidx_f, ig_hbm)):
        pltpu.sync_copy(idx_f, ix_v.at[pl.ds(0, B)])

        def issue(jn, buf, sm, tab=tab, last=last):
            jc = jnp.minimum(jn, NBLK - 2)

            @pl.when(jn != LASTB)
            def _():
                pltpu.async_copy(
                    tab.at[:, pl.ds(pl.multiple_of(jc * 128, 128), 128)],
                    buf, sm)

            @pl.when(jn == LASTB)
            def _():
                pltpu.async_copy(last, buf, sm)

        def fbody(j, ct):
            v = ix_v[pl.ds(j * L, L)]
            m = jnp.logical_and(v >= lo_r, v < hi_r)
            plsc.store_compressed(wl_r.at[pl.ds(ct, L)], v, mask=m)
            plsc.store_compressed(wl_p.at[pl.ds(ct, L)], j * L + lanes, mask=m)
            return ct + plsc.all_reduce_population_count(m)[0]

        n_wl = lax.fori_loop(0, B // L, fbody, 0)

        wl2_r = ix_v
        wl2_p = hit_p
        zeros16 = jnp.zeros((L,), jnp.int32)
        off_v[pl.ds(32, L)] = zeros16

        def cbody(i, _):
            iv = jnp.full((L,), i, jnp.int32)
            rv = plsc.load_gather(wl_r, [iv])[0]
            bkt = lax.shift_right_logical(
                lax.shift_right_logical(rv, 7) - lo_blk, 4)
            bv = jnp.full((L,), bkt, jnp.int32)
            h = plsc.load_gather(off_v, [32 + bv])[0]
            plsc.store_scatter(off_v, [32 + bv],
                               jnp.full((L,), h + 1, jnp.int32), mask=lane0)
            return 0

        lax.fori_loop(0, n_wl, cbody, 0)
        hist = off_v[pl.ds(32, L)]
        incl = plsc.cumsum(hist)
        off_v[pl.ds(0, L)] = incl - hist
        off_v[pl.ds(16, L)] = incl
        off_v[pl.ds(32, L)] = incl - hist

        def pbody(i, _):
            iv = jnp.full((L,), i, jnp.int32)
            rv = plsc.load_gather(wl_r, [iv])[0]
            pv = plsc.load_gather(wl_p, [iv])[0]
            bkt = lax.shift_right_logical(
                lax.shift_right_logical(rv, 7) - lo_blk, 4)
            bv = jnp.full((L,), bkt, jnp.int32)
            cur = plsc.load_gather(off_v, [32 + bv])[0]
            cv = jnp.full((L,), cur, jnp.int32)
            plsc.store_scatter(wl2_r, [cv], jnp.full((L,), rv, jnp.int32),
                               mask=lane0)
            plsc.store_scatter(wl2_p, [cv], jnp.full((L,), pv, jnp.int32),
                               mask=lane0)
            plsc.store_scatter(off_v, [32 + bv],
                               jnp.full((L,), cur + 1, jnp.int32), mask=lane0)
            return 0

        lax.fori_loop(0, n_wl, pbody, 0)
        hit_cb = wl_r
        hit_pb = wl_p

        nb = hi_blk - lo_blk
        nb4 = lax.shift_left(lax.shift_right_logical(nb + 3, 2), 2)
        for p in range(3):
            issue(lo_blk + p, bufs[p], sems[p])

        def bbody(q, st):
            for s in range(4):
                j = lo_blk + 4 * q + s
                buf = bufs[s]

                jn = j + 3

                @pl.when(jn < lo_blk + nb4)
                def _(jn=jn, s=s):
                    issue(jn, bufs[(s + 3) % 4], sems[(s + 3) % 4])

                bkt = lax.shift_right_logical(j - lo_blk, 4)
                bv = jnp.full((L,), bkt, jnp.int32)
                bk_lo = plsc.load_gather(off_v, [bv])[0]
                bk_hi = plsc.load_gather(off_v, [16 + bv])[0]
                k_lo = lax.shift_right_logical(bk_lo, 4)
                k_hi = lax.shift_right_logical(bk_hi + 15, 4)

                def sbody(k, hc, j=j):
                    base = k * L
                    rv = wl2_r[pl.ds(pl.multiple_of(base, L), L)]
                    pv = wl2_p[pl.ds(pl.multiple_of(base, L), L)]
                    m = jnp.logical_and(
                        jnp.logical_and(base + lanes >= bk_lo,
                                        base + lanes < bk_hi),
                        jnp.logical_and(rv >= j * 128, rv < j * 128 + 128))
                    col = jnp.where(j == LASTB, rv - (N - 128), rv - j * 128)
                    plsc.store_compressed(hit_cb.at[pl.ds(hc, L)], col, mask=m)
                    plsc.store_compressed(hit_pb.at[pl.ds(hc, L)], pv, mask=m)
                    return hc + plsc.all_reduce_population_count(m)[0]

                nh = lax.fori_loop(k_lo, k_hi, sbody, 0)

                pltpu.make_async_copy(
                    tab.at[:, pl.ds(0, 128)], buf, sems[s]).wait()

                def hbody(i, st_in, buf=buf):
                    iv = jnp.full((L,), i, jnp.int32)
                    c = plsc.load_gather(hit_cb, [iv])[0]
                    pv = plsc.load_gather(hit_pb, [iv])[0]
                    plsc.store_scatter(pos_st, [jnp.full((L,), st_in, jnp.int32)],
                                       jnp.full((L,), pv, jnp.int32), mask=lane0)
                    cv = jnp.full((L,), c, jnp.int32)
                    for k in range(D // L):
                        stg_v[st_in, pl.ds(k * L, L)] = plsc.load_gather(
                            buf, [k * L + lanes, cv])
                    st2 = st_in + 1

                    @pl.when(st2 == 128)
                    def _():
                        pltpu.async_copy(stg_v, og.at[pos_st], sem).wait()
                    return jnp.where(st2 == 128, 0, st2)

                st = lax.fori_loop(0, nh, hbody, st)
            return st

        st_end = lax.fori_loop(0, lax.shift_right_logical(nb4, 2), bbody, 0)

        def padbody(i, _):
            plsc.store_scatter(pos_st, [jnp.full((L,), st_end + i, jnp.int32)],
                               jnp.full((L,), trash, jnp.int32), mask=lane0)
            return 0

        @pl.when(st_end > 0)
        def _():
            lax.fori_loop(0, 128 - st_end, padbody, 0)
            pltpu.async_copy(stg_v, og.at[pos_st], sem).wait()


def _dot_body(ug_hbm, ig_hbm, uidx2, iidx2, ub_hbm, ib_hbm, out_hbm,
              idx_v, bb_v, ue_v, ie_v, out_v, sem):
    wid = lax.axis_index("s") * NC + lax.axis_index("c")
    pltpu.sync_copy(uidx2.at[pl.ds(wid * 4, 4)], idx_v.at[pl.ds(0, 4)])
    pltpu.sync_copy(iidx2.at[pl.ds(wid * 4, 4)], idx_v.at[pl.ds(4, 4)])
    descs = []
    for c in range(4):
        descs.append(pltpu.async_copy(ub_hbm.at[idx_v.at[c]], bb_v.at[c], sem))
        descs.append(pltpu.async_copy(ib_hbm.at[idx_v.at[c + 4]], bb_v.at[c + 4], sem))

    lanes = lax.iota(jnp.int32, L)
    for h in range(2):
        row0 = wid * BPW + h * 256
        d1 = pltpu.async_copy(ug_hbm.at[pl.ds(row0, 256)], ue_v, sem)
        d2 = pltpu.async_copy(ig_hbm.at[pl.ds(row0, 256)], ie_v, sem)
        d1.wait()
        d2.wait()

        def grp_body(jj, _, h=h):
            vec = jnp.zeros((L,), jnp.float32)
            for t in range(L):
                lr = jj * L + t
                acc = ue_v[lr, pl.ds(0, L)] * ie_v[lr, pl.ds(0, L)]
                for k in range(1, D // L):
                    acc = acc + (ue_v[lr, pl.ds(k * L, L)]
                                 * ie_v[lr, pl.ds(k * L, L)])
                vec = jnp.where(lanes == t, jnp.sum(acc), vec)
            out_v[pl.ds(h * 256 + jj * L, L)] = vec
            return 0

        lax.fori_loop(0, 256 // L, grp_body, 0)

    for d_ in descs:
        d_.wait()
    for c in range(4):
        for j in range(128 // L):
            s = pl.ds(j * L, L)
            r = out_v[pl.ds(c * 128 + j * L, L)] + bb_v[c, s] + bb_v[c + 4, s]
            out_v[pl.ds(c * 128 + j * L, L)] = 5.0 / (1.0 + jnp.exp(-r))
    pltpu.sync_copy(out_v, out_hbm.at[pl.ds(wid * BPW, BPW)])


def kernel(x_batch, user_emb, item_emb, user_bias, item_bias):
    ue_t = user_emb.T
    ie_t = item_emb.T
    ulast = lax.slice(ue_t, (0, N - 128), (D, N))
    ilast = lax.slice(ie_t, (0, N - 128), (D, N))
    ub = user_bias.reshape(-1)
    ib = item_bias.reshape(-1)
    uidx2 = x_batch[:, 0].reshape(B // 128, 128)
    iidx2 = x_batch[:, 1].reshape(B // 128, 128)

    mesh = plsc.VectorSubcoreMesh(core_axis_name="c", subcore_axis_name="s")

    gather_k = pl.kernel(
        _gather_body,
        out_type=(jax.ShapeDtypeStruct((GOUT, 128), jnp.float32),
                  jax.ShapeDtypeStruct((GOUT, 128), jnp.float32)),
        mesh=mesh,
        compiler_params=pltpu.CompilerParams(
            needs_layout_passes=False, use_tc_tiling_on_sc=True
        ),
        scratch_types=[
            pltpu.VMEM((WLCAP,), jnp.int32),
            pltpu.VMEM((WLCAP,), jnp.int32),
            pltpu.VMEM((WLCAP,), jnp.int32),
            pltpu.VMEM((WLCAP,), jnp.int32),
            pltpu.VMEM((48,), jnp.int32),
            pltpu.VMEM((D, 128), jnp.float32),
            pltpu.VMEM((D, 128), jnp.float32),
            pltpu.VMEM((D, 128), jnp.float32),
            pltpu.VMEM((D, 128), jnp.float32),
            pltpu.VMEM((128, 128), jnp.float32),
            pltpu.VMEM((128,), jnp.int32),
            pltpu.SemaphoreType.DMA,
            pltpu.SemaphoreType.DMA,
            pltpu.SemaphoreType.DMA,
            pltpu.SemaphoreType.DMA,
            pltpu.SemaphoreType.DMA,
        ],
    )
    ug, ig = gather_k(ue_t, ie_t, ulast, ilast,
                      x_batch[:, 0], x_batch[:, 1])

    dot_k = pl.kernel(
        _dot_body,
        out_type=jax.ShapeDtypeStruct((B,), jnp.float32),
        mesh=mesh,
        compiler_params=pltpu.CompilerParams(
            needs_layout_passes=False, use_tc_tiling_on_sc=False
        ),
        scratch_types=[
            pltpu.VMEM((8, 128), jnp.int32),
            pltpu.VMEM((8, 128), jnp.float32),
            pltpu.VMEM((256, 128), jnp.float32),
            pltpu.VMEM((256, 128), jnp.float32),
            pltpu.VMEM((BPW,), jnp.float32),
            pltpu.SemaphoreType.DMA,
        ],
    )
    return dot_k(ug, ig, uidx2, iidx2, ub, ib)


# speedup vs baseline: 1.0506x; 1.0506x over previous
import jax
import jax.numpy as jnp
from jax import lax
from jax.experimental import pallas as pl
from jax.experimental.pallas import tpu as pltpu
from jax.experimental.pallas import tpu_sc as plsc

B = 16384
D = 64
N = 1000000
NC = 2
NS = 16
NW = NC * NS
BPW = B // NW
L = 16
NBLK = (N + 127) // 128
LASTB = NBLK - 1
GOUT = B + NW
WLCAP = B + L


def _gather_body(uemb_t, iemb_t, ulast, ilast, uidx_f, iidx_f,
                 ug_hbm, ig_hbm,
                 ix_v, wl_r, wl_p, hit_p, off_v, ne_v,
                 blk0, blk1, blk2, blk3,
                 stg_v, pos_st,
                 sem, sem0, sem1, sem2, sem3):
    wid = lax.axis_index("s") * NC + lax.axis_index("c")
    lo_blk = lax.shift_right_logical(wid * NBLK, 5)
    hi_blk = lax.shift_right_logical((wid + 1) * NBLK, 5)
    lo_r = lo_blk * 128
    hi_r = hi_blk * 128
    lanes = lax.iota(jnp.int32, L)
    lane0 = lanes == 0
    trash = B + wid
    bufs = (blk0, blk1, blk2, blk3)
    sems = (sem0, sem1, sem2, sem3)

    for tab, last, idx_f, og in ((uemb_t, ulast, uidx_f, ug_hbm),
                                 (iemb_t, ilast, iidx_f, ig_hbm)):
        pltpu.sync_copy(idx_f, ix_v.at[pl.ds(0, B)])

        def issue(jn, buf, sm, tab=tab, last=last):
            jc = jnp.minimum(jn, NBLK - 2)

            @pl.when(jn != LASTB)
            def _():
                pltpu.async_copy(
                    tab.at[:, pl.ds(pl.multiple_of(jc * 128, 128), 128)],
                    buf, sm)

            @pl.when(jn == LASTB)
            def _():
                pltpu.async_copy(last, buf, sm)

        def fbody(j, ct):
            v = ix_v[pl.ds(j * L, L)]
            m = jnp.logical_and(v >= lo_r, v < hi_r)
            plsc.store_compressed(wl_r.at[pl.ds(ct, L)], v, mask=m)
            plsc.store_compressed(wl_p.at[pl.ds(ct, L)], j * L + lanes, mask=m)
            return ct + plsc.all_reduce_population_count(m)[0]

        n_wl = lax.fori_loop(0, B // L, fbody, 0)

        wl2_r = ix_v
        wl2_p = hit_p
        zeros16 = jnp.zeros((L,), jnp.int32)
        for kk in range(16):
            off_v[pl.ds(512 + kk * L, L)] = zeros16

        def cbody(i, _):
            iv = jnp.full((L,), i, jnp.int32)
            rv = plsc.load_gather(wl_r, [iv])[0]
            bl = lax.shift_right_logical(rv, 7) - lo_blk
            bv = jnp.full((L,), bl, jnp.int32)
            h = plsc.load_gather(off_v, [512 + bv])[0]
            plsc.store_scatter(off_v, [512 + bv],
                               jnp.full((L,), h + 1, jnp.int32), mask=lane0)
            return 0

        lax.fori_loop(0, n_wl, cbody, 0)

        def csum(kk, carry):
            h = off_v[pl.ds(pl.multiple_of(512 + kk * L, L), L)]
            incl = plsc.cumsum(h) + carry
            off_v[pl.ds(pl.multiple_of(kk * L, L), L)] = incl - h
            off_v[pl.ds(pl.multiple_of(256 + kk * L, L), L)] = incl
            off_v[pl.ds(pl.multiple_of(512 + kk * L, L), L)] = incl - h
            return incl[15]

        lax.fori_loop(0, 16, csum, 0)

        def pbody(i, _):
            iv = jnp.full((L,), i, jnp.int32)
            rv = plsc.load_gather(wl_r, [iv])[0]
            pv = plsc.load_gather(wl_p, [iv])[0]
            bl = lax.shift_right_logical(rv, 7) - lo_blk
            bv = jnp.full((L,), bl, jnp.int32)
            cur = plsc.load_gather(off_v, [512 + bv])[0]
            cv = jnp.full((L,), cur, jnp.int32)
            plsc.store_scatter(wl2_r, [cv], jnp.full((L,), rv, jnp.int32),
                               mask=lane0)
            plsc.store_scatter(wl2_p, [cv], jnp.full((L,), pv, jnp.int32),
                               mask=lane0)
            plsc.store_scatter(off_v, [512 + bv],
                               jnp.full((L,), cur + 1, jnp.int32), mask=lane0)
            return 0

        lax.fori_loop(0, n_wl, pbody, 0)

        nb = hi_blk - lo_blk

        def nebody(kk, ct):
            base = kk * L
            excl = off_v[pl.ds(pl.multiple_of(base, L), L)]
            incl = off_v[pl.ds(pl.multiple_of(256 + base, L), L)]
            m = jnp.logical_and(incl > excl, base + lanes < nb)
            plsc.store_compressed(ne_v.at[pl.ds(ct, L)],
                                  lo_blk + base + lanes, mask=m)
            return ct + plsc.all_reduce_population_count(m)[0]

        ne = lax.fori_loop(0, 16, nebody, 0)
        ne4 = lax.shift_left(lax.shift_right_logical(ne + 3, 2), 2)
        last_ne = plsc.load_gather(
            ne_v, [jnp.full((L,), jnp.maximum(ne - 1, 0), jnp.int32)])[0]

        def padne(p, _):
            plsc.store_scatter(ne_v, [jnp.full((L,), ne + p, jnp.int32)],
                               jnp.full((L,), last_ne, jnp.int32), mask=lane0)
            return 0

        lax.fori_loop(0, ne4 - ne, padne, 0)

        def blk_at(i):
            return plsc.load_gather(ne_v, [jnp.full((L,), i, jnp.int32)])[0]

        for p in range(3):
            @pl.when(p < ne4)
            def _(p=p):
                issue(blk_at(p), bufs[p], sems[p])

        def bbody(q, st):
            for s in range(4):
                i_blk = 4 * q + s
                j = blk_at(i_blk)
                buf = bufs[s]

                @pl.when(i_blk + 3 < ne4)
                def _(i_blk=i_blk, s=s):
                    issue(blk_at(i_blk + 3), bufs[(s + 3) % 4],
                          sems[(s + 3) % 4])

                bl = j - lo_blk
                bv = jnp.full((L,), bl, jnp.int32)
                bk_lo = plsc.load_gather(off_v, [bv])[0]
                bk_hi = plsc.load_gather(off_v, [256 + bv])[0]

                pltpu.make_async_copy(
                    tab.at[:, pl.ds(0, 128)], buf, sems[s]).wait()

                def hbody(i, st_in, buf=buf, j=j):
                    iv = jnp.full((L,), i, jnp.int32)
                    rv = plsc.load_gather(wl2_r, [iv])[0]
                    pv = plsc.load_gather(wl2_p, [iv])[0]
                    c = jnp.where(j == LASTB, rv - (N - 128), rv - j * 128)
                    plsc.store_scatter(pos_st, [jnp.full((L,), st_in, jnp.int32)],
                                       jnp.full((L,), pv, jnp.int32), mask=lane0)
                    cv = jnp.full((L,), c, jnp.int32)
                    for k in range(D // L):
                        stg_v[st_in, pl.ds(k * L, L)] = plsc.load_gather(
                            buf, [k * L + lanes, cv])
                    st2 = st_in + 1

                    @pl.when(st2 == 128)
                    def _():
                        pltpu.async_copy(stg_v, og.at[pos_st], sem).wait()
                    return jnp.where(st2 == 128, 0, st2)

                st = lax.fori_loop(bk_lo, bk_hi, hbody, st)
            return st

        st_end = lax.fori_loop(0, lax.shift_right_logical(ne4, 2), bbody, 0)

        def padbody(i, _):
            plsc.store_scatter(pos_st, [jnp.full((L,), st_end + i, jnp.int32)],
                               jnp.full((L,), trash, jnp.int32), mask=lane0)
            return 0

        @pl.when(st_end > 0)
        def _():
            lax.fori_loop(0, 128 - st_end, padbody, 0)
            pltpu.async_copy(stg_v, og.at[pos_st], sem).wait()


def _dot_body(ug_hbm, ig_hbm, uidx2, iidx2, ub_hbm, ib_hbm, out_hbm,
              idx_v, bb_v, ue_v, ie_v, out_v, sem):
    wid = lax.axis_index("s") * NC + lax.axis_index("c")
    pltpu.sync_copy(uidx2.at[pl.ds(wid * 4, 4)], idx_v.at[pl.ds(0, 4)])
    pltpu.sync_copy(iidx2.at[pl.ds(wid * 4, 4)], idx_v.at[pl.ds(4, 4)])
    descs = []
    for c in range(4):
        descs.append(pltpu.async_copy(ub_hbm.at[idx_v.at[c]], bb_v.at[c], sem))
        descs.append(pltpu.async_copy(ib_hbm.at[idx_v.at[c + 4]], bb_v.at[c + 4], sem))

    lanes = lax.iota(jnp.int32, L)
    for h in range(2):
        row0 = wid * BPW + h * 256
        d1 = pltpu.async_copy(ug_hbm.at[pl.ds(row0, 256)], ue_v, sem)
        d2 = pltpu.async_copy(ig_hbm.at[pl.ds(row0, 256)], ie_v, sem)
        d1.wait()
        d2.wait()

        def grp_body(jj, _, h=h):
            vec = jnp.zeros((L,), jnp.float32)
            for t in range(L):
                lr = jj * L + t
                acc = ue_v[lr, pl.ds(0, L)] * ie_v[lr, pl.ds(0, L)]
                for k in range(1, D // L):
                    acc = acc + (ue_v[lr, pl.ds(k * L, L)]
                                 * ie_v[lr, pl.ds(k * L, L)])
                vec = jnp.where(lanes == t, jnp.sum(acc), vec)
            out_v[pl.ds(h * 256 + jj * L, L)] = vec
            return 0

        lax.fori_loop(0, 256 // L, grp_body, 0)

    for d_ in descs:
        d_.wait()
    for c in range(4):
        for j in range(128 // L):
            s = pl.ds(j * L, L)
            r = out_v[pl.ds(c * 128 + j * L, L)] + bb_v[c, s] + bb_v[c + 4, s]
            out_v[pl.ds(c * 128 + j * L, L)] = 5.0 / (1.0 + jnp.exp(-r))
    pltpu.sync_copy(out_v, out_hbm.at[pl.ds(wid * BPW, BPW)])


def kernel(x_batch, user_emb, item_emb, user_bias, item_bias):
    ue_t = user_emb.T
    ie_t = item_emb.T
    ulast = lax.slice(ue_t, (0, N - 128), (D, N))
    ilast = lax.slice(ie_t, (0, N - 128), (D, N))
    ub = user_bias.reshape(-1)
    ib = item_bias.reshape(-1)
    uidx2 = x_batch[:, 0].reshape(B // 128, 128)
    iidx2 = x_batch[:, 1].reshape(B // 128, 128)

    mesh = plsc.VectorSubcoreMesh(core_axis_name="c", subcore_axis_name="s")

    gather_k = pl.kernel(
        _gather_body,
        out_type=(jax.ShapeDtypeStruct((GOUT, 128), jnp.float32),
                  jax.ShapeDtypeStruct((GOUT, 128), jnp.float32)),
        mesh=mesh,
        compiler_params=pltpu.CompilerParams(
            needs_layout_passes=False, use_tc_tiling_on_sc=True
        ),
        scratch_types=[
            pltpu.VMEM((WLCAP,), jnp.int32),
            pltpu.VMEM((WLCAP,), jnp.int32),
            pltpu.VMEM((WLCAP,), jnp.int32),
            pltpu.VMEM((WLCAP,), jnp.int32),
            pltpu.VMEM((768,), jnp.int32),
            pltpu.VMEM((256,), jnp.int32),
            pltpu.VMEM((D, 128), jnp.float32),
            pltpu.VMEM((D, 128), jnp.float32),
            pltpu.VMEM((D, 128), jnp.float32),
            pltpu.VMEM((D, 128), jnp.float32),
            pltpu.VMEM((128, 128), jnp.float32),
            pltpu.VMEM((128,), jnp.int32),
            pltpu.SemaphoreType.DMA,
            pltpu.SemaphoreType.DMA,
            pltpu.SemaphoreType.DMA,
            pltpu.SemaphoreType.DMA,
            pltpu.SemaphoreType.DMA,
        ],
    )
    ug, ig = gather_k(ue_t, ie_t, ulast, ilast,
                      x_batch[:, 0], x_batch[:, 1])

    dot_k = pl.kernel(
        _dot_body,
        out_type=jax.ShapeDtypeStruct((B,), jnp.float32),
        mesh=mesh,
        compiler_params=pltpu.CompilerParams(
            needs_layout_passes=False, use_tc_tiling_on_sc=False
        ),
        scratch_types=[
            pltpu.VMEM((8, 128), jnp.int32),
            pltpu.VMEM((8, 128), jnp.float32),
            pltpu.VMEM((256, 128), jnp.float32),
            pltpu.VMEM((256, 128), jnp.float32),
            pltpu.VMEM((BPW,), jnp.float32),
            pltpu.SemaphoreType.DMA,
        ],
    )
    return dot_k(ug, ig, uidx2, iidx2, ub, ib)


# speedup vs baseline: 1.1029x; 1.0498x over previous
import jax
import jax.numpy as jnp
from jax import lax
from jax.experimental import pallas as pl
from jax.experimental.pallas import tpu as pltpu
from jax.experimental.pallas import tpu_sc as plsc

B = 16384
D = 64
N = 1000000
NC = 2
NS = 16
NW = NC * NS
BPW = B // NW
L = 16
NBLK = (N + 127) // 128
LASTB = NBLK - 1
GOUT = B + NW
WLCAP = B + L


def _gather_body(uemb_t, iemb_t, ulast, ilast, uidx_f, iidx_f,
                 ug_hbm, ig_hbm,
                 ix_v, wl_r, wl_p, hit_p, off_v, ne_v,
                 blk0, blk1, blk2, blk3, blk4,
                 stg_v, pos_st,
                 sem, sem0, sem1, sem2, sem3, sem4):
    wid = lax.axis_index("s") * NC + lax.axis_index("c")
    lo_blk = lax.shift_right_logical(wid * NBLK, 5)
    hi_blk = lax.shift_right_logical((wid + 1) * NBLK, 5)
    lo_r = lo_blk * 128
    hi_r = hi_blk * 128
    lanes = lax.iota(jnp.int32, L)
    lane0 = lanes == 0
    trash = B + wid
    bufs = (blk0, blk1, blk2, blk3, blk4)
    sems = (sem0, sem1, sem2, sem3, sem4)

    for tab, last, idx_f, og in ((uemb_t, ulast, uidx_f, ug_hbm),
                                 (iemb_t, ilast, iidx_f, ig_hbm)):
        pltpu.sync_copy(idx_f, ix_v.at[pl.ds(0, B)])

        def issue(jn, buf, sm, tab=tab, last=last):
            jc = jnp.minimum(jn, NBLK - 2)

            @pl.when(jn != LASTB)
            def _():
                pltpu.async_copy(
                    tab.at[:, pl.ds(pl.multiple_of(jc * 128, 128), 128)],
                    buf, sm)

            @pl.when(jn == LASTB)
            def _():
                pltpu.async_copy(last, buf, sm)

        def fbody(j, ct):
            v = ix_v[pl.ds(j * L, L)]
            m = jnp.logical_and(v >= lo_r, v < hi_r)
            plsc.store_compressed(wl_r.at[pl.ds(ct, L)], v, mask=m)
            plsc.store_compressed(wl_p.at[pl.ds(ct, L)], j * L + lanes, mask=m)
            return ct + plsc.all_reduce_population_count(m)[0]

        n_wl = lax.fori_loop(0, B // L, fbody, 0)

        wl2_r = ix_v
        wl2_p = hit_p
        zeros16 = jnp.zeros((L,), jnp.int32)
        for kk in range(16):
            off_v[pl.ds(512 + kk * L, L)] = zeros16

        def cbody(i, _):
            iv = jnp.full((L,), i, jnp.int32)
            rv = plsc.load_gather(wl_r, [iv])[0]
            bl = lax.shift_right_logical(rv, 7) - lo_blk
            bv = jnp.full((L,), bl, jnp.int32)
            h = plsc.load_gather(off_v, [512 + bv])[0]
            plsc.store_scatter(off_v, [512 + bv],
                               jnp.full((L,), h + 1, jnp.int32), mask=lane0)
            return 0

        lax.fori_loop(0, n_wl, cbody, 0)

        def csum(kk, carry):
            h = off_v[pl.ds(pl.multiple_of(512 + kk * L, L), L)]
            incl = plsc.cumsum(h) + carry
            off_v[pl.ds(pl.multiple_of(kk * L, L), L)] = incl - h
            off_v[pl.ds(pl.multiple_of(256 + kk * L, L), L)] = incl
            off_v[pl.ds(pl.multiple_of(512 + kk * L, L), L)] = incl - h
            return incl[15]

        lax.fori_loop(0, 16, csum, 0)

        def pbody(i, _):
            iv = jnp.full((L,), i, jnp.int32)
            rv = plsc.load_gather(wl_r, [iv])[0]
            pv = plsc.load_gather(wl_p, [iv])[0]
            bl = lax.shift_right_logical(rv, 7) - lo_blk
            bv = jnp.full((L,), bl, jnp.int32)
            cur = plsc.load_gather(off_v, [512 + bv])[0]
            cv = jnp.full((L,), cur, jnp.int32)
            plsc.store_scatter(wl2_r, [cv], jnp.full((L,), rv, jnp.int32),
                               mask=lane0)
            plsc.store_scatter(wl2_p, [cv], jnp.full((L,), pv, jnp.int32),
                               mask=lane0)
            plsc.store_scatter(off_v, [512 + bv],
                               jnp.full((L,), cur + 1, jnp.int32), mask=lane0)
            return 0

        lax.fori_loop(0, n_wl, pbody, 0)

        nb = hi_blk - lo_blk

        def nebody(kk, ct):
            base = kk * L
            excl = off_v[pl.ds(pl.multiple_of(base, L), L)]
            incl = off_v[pl.ds(pl.multiple_of(256 + base, L), L)]
            m = jnp.logical_and(incl > excl, base + lanes < nb)
            plsc.store_compressed(ne_v.at[pl.ds(ct, L)],
                                  lo_blk + base + lanes, mask=m)
            return ct + plsc.all_reduce_population_count(m)[0]

        ne = lax.fori_loop(0, 16, nebody, 0)
        ne4 = lax.div(ne + 4, 5) * 5
        last_ne = plsc.load_gather(
            ne_v, [jnp.full((L,), jnp.maximum(ne - 1, 0), jnp.int32)])[0]

        def padne(p, _):
            plsc.store_scatter(ne_v, [jnp.full((L,), ne + p, jnp.int32)],
                               jnp.full((L,), last_ne, jnp.int32), mask=lane0)
            return 0

        lax.fori_loop(0, ne4 - ne, padne, 0)

        def blk_at(i):
            return plsc.load_gather(ne_v, [jnp.full((L,), i, jnp.int32)])[0]

        for p in range(4):
            @pl.when(p < ne4)
            def _(p=p):
                issue(blk_at(p), bufs[p], sems[p])

        def bbody(q, st):
            for s in range(5):
                i_blk = 5 * q + s
                j = blk_at(i_blk)
                buf = bufs[s]

                @pl.when(i_blk + 4 < ne4)
                def _(i_blk=i_blk, s=s):
                    issue(blk_at(i_blk + 4), bufs[(s + 4) % 5],
                          sems[(s + 4) % 5])

                bl = j - lo_blk
                bv = jnp.full((L,), bl, jnp.int32)
                bk_lo = plsc.load_gather(off_v, [bv])[0]
                bk_hi = plsc.load_gather(off_v, [256 + bv])[0]

                pltpu.make_async_copy(
                    tab.at[:, pl.ds(0, 128)], buf, sems[s]).wait()

                def hbody(i, st_in, buf=buf, j=j):
                    iv = jnp.full((L,), i, jnp.int32)
                    rv = plsc.load_gather(wl2_r, [iv])[0]
                    pv = plsc.load_gather(wl2_p, [iv])[0]
                    c = jnp.where(j == LASTB, rv - (N - 128), rv - j * 128)
                    plsc.store_scatter(pos_st, [jnp.full((L,), st_in, jnp.int32)],
                                       jnp.full((L,), pv, jnp.int32), mask=lane0)
                    cv = jnp.full((L,), c, jnp.int32)
                    for k in range(D // L):
                        stg_v[st_in, pl.ds(k * L, L)] = plsc.load_gather(
                            buf, [k * L + lanes, cv])
                    st2 = st_in + 1

                    @pl.when(st2 == 128)
                    def _():
                        pltpu.async_copy(stg_v, og.at[pos_st], sem).wait()
                    return jnp.where(st2 == 128, 0, st2)

                st = lax.fori_loop(bk_lo, bk_hi, hbody, st)
            return st

        st_end = lax.fori_loop(0, lax.div(ne4, 5), bbody, 0)

        def padbody(i, _):
            plsc.store_scatter(pos_st, [jnp.full((L,), st_end + i, jnp.int32)],
                               jnp.full((L,), trash, jnp.int32), mask=lane0)
            return 0

        @pl.when(st_end > 0)
        def _():
            lax.fori_loop(0, 128 - st_end, padbody, 0)
            pltpu.async_copy(stg_v, og.at[pos_st], sem).wait()


def _dot_body(ug_hbm, ig_hbm, uidx2, iidx2, ub_hbm, ib_hbm, out_hbm,
              idx_v, bb_v, ue_v, ie_v, out_v, sem):
    wid = lax.axis_index("s") * NC + lax.axis_index("c")
    pltpu.sync_copy(uidx2.at[pl.ds(wid * 4, 4)], idx_v.at[pl.ds(0, 4)])
    pltpu.sync_copy(iidx2.at[pl.ds(wid * 4, 4)], idx_v.at[pl.ds(4, 4)])
    descs = []
    for c in range(4):
        descs.append(pltpu.async_copy(ub_hbm.at[idx_v.at[c]], bb_v.at[c], sem))
        descs.append(pltpu.async_copy(ib_hbm.at[idx_v.at[c + 4]], bb_v.at[c + 4], sem))

    lanes = lax.iota(jnp.int32, L)
    for h in range(2):
        row0 = wid * BPW + h * 256
        d1 = pltpu.async_copy(ug_hbm.at[pl.ds(row0, 256)], ue_v, sem)
        d2 = pltpu.async_copy(ig_hbm.at[pl.ds(row0, 256)], ie_v, sem)
        d1.wait()
        d2.wait()

        def grp_body(jj, _, h=h):
            vec = jnp.zeros((L,), jnp.float32)
            for t in range(L):
                lr = jj * L + t
                acc = ue_v[lr, pl.ds(0, L)] * ie_v[lr, pl.ds(0, L)]
                for k in range(1, D // L):
                    acc = acc + (ue_v[lr, pl.ds(k * L, L)]
                                 * ie_v[lr, pl.ds(k * L, L)])
                vec = jnp.where(lanes == t, jnp.sum(acc), vec)
            out_v[pl.ds(h * 256 + jj * L, L)] = vec
            return 0

        lax.fori_loop(0, 256 // L, grp_body, 0)

    for d_ in descs:
        d_.wait()
    for c in range(4):
        for j in range(128 // L):
            s = pl.ds(j * L, L)
            r = out_v[pl.ds(c * 128 + j * L, L)] + bb_v[c, s] + bb_v[c + 4, s]
            out_v[pl.ds(c * 128 + j * L, L)] = 5.0 / (1.0 + jnp.exp(-r))
    pltpu.sync_copy(out_v, out_hbm.at[pl.ds(wid * BPW, BPW)])


def kernel(x_batch, user_emb, item_emb, user_bias, item_bias):
    ue_t = user_emb.T
    ie_t = item_emb.T
    ulast = lax.slice(ue_t, (0, N - 128), (D, N))
    ilast = lax.slice(ie_t, (0, N - 128), (D, N))
    ub = user_bias.reshape(-1)
    ib = item_bias.reshape(-1)
    uidx2 = x_batch[:, 0].reshape(B // 128, 128)
    iidx2 = x_batch[:, 1].reshape(B // 128, 128)

    mesh = plsc.VectorSubcoreMesh(core_axis_name="c", subcore_axis_name="s")

    gather_k = pl.kernel(
        _gather_body,
        out_type=(jax.ShapeDtypeStruct((GOUT, 128), jnp.float32),
                  jax.ShapeDtypeStruct((GOUT, 128), jnp.float32)),
        mesh=mesh,
        compiler_params=pltpu.CompilerParams(
            needs_layout_passes=False, use_tc_tiling_on_sc=True
        ),
        scratch_types=[
            pltpu.VMEM((WLCAP,), jnp.int32),
            pltpu.VMEM((WLCAP,), jnp.int32),
            pltpu.VMEM((WLCAP,), jnp.int32),
            pltpu.VMEM((WLCAP,), jnp.int32),
            pltpu.VMEM((768,), jnp.int32),
            pltpu.VMEM((256,), jnp.int32),
            pltpu.VMEM((D, 128), jnp.float32),
            pltpu.VMEM((D, 128), jnp.float32),
            pltpu.VMEM((D, 128), jnp.float32),
            pltpu.VMEM((D, 128), jnp.float32),
            pltpu.VMEM((D, 128), jnp.float32),
            pltpu.VMEM((128, 128), jnp.float32),
            pltpu.VMEM((128,), jnp.int32),
            pltpu.SemaphoreType.DMA,
            pltpu.SemaphoreType.DMA,
            pltpu.SemaphoreType.DMA,
            pltpu.SemaphoreType.DMA,
            pltpu.SemaphoreType.DMA,
            pltpu.SemaphoreType.DMA,
        ],
    )
    ug, ig = gather_k(ue_t, ie_t, ulast, ilast,
                      x_batch[:, 0], x_batch[:, 1])

    dot_k = pl.kernel(
        _dot_body,
        out_type=jax.ShapeDtypeStruct((B,), jnp.float32),
        mesh=mesh,
        compiler_params=pltpu.CompilerParams(
            needs_layout_passes=False, use_tc_tiling_on_sc=False
        ),
        scratch_types=[
            pltpu.VMEM((8, 128), jnp.int32),
            pltpu.VMEM((8, 128), jnp.float32),
            pltpu.VMEM((256, 128), jnp.float32),
            pltpu.VMEM((256, 128), jnp.float32),
            pltpu.VMEM((BPW,), jnp.float32),
            pltpu.SemaphoreType.DMA,
        ],
    )
    return dot_k(ug, ig, uidx2, iidx2, ub, ib)
